# baseline (speedup 1.0000x reference)
import functools

import jax
import jax.numpy as jnp
from jax import lax
from jax.experimental import pallas as pl
from jax.experimental.pallas import tpu as pltpu
from jax.experimental.pallas import tpu_sc as plsc

_TILE = 512
_BLK = 128
_K = 2


def _router_body(x_ref, wg_ref, pos_ref, gw_ref,
                 exp_ref, xgi_ref, valid_ref):
    x = x_ref[...]
    wg = wg_ref[...]
    logits = lax.dot_general(x, wg, (((1,), (1,)), ((), ())),
                             preferred_element_type=jnp.float32)
    s, e = logits.shape
    eio = lax.broadcasted_iota(jnp.int32, (s, e), 1)
    m1 = jnp.max(logits, axis=1, keepdims=True)
    i1 = jnp.min(jnp.where(logits == m1, eio, e), axis=1, keepdims=True)
    l2 = jnp.where(eio == i1, -jnp.inf, logits)
    m2 = jnp.max(l2, axis=1, keepdims=True)
    i2 = jnp.min(jnp.where(l2 == m2, eio, e), axis=1, keepdims=True)
    w1 = 1.0 / (1.0 + jnp.exp(m2 - m1))

    oh0 = (eio == i1).astype(jnp.float32)
    oh1 = (eio == i2).astype(jnp.float32)
    oh = oh0 + oh1

    rio = lax.broadcasted_iota(jnp.int32, (_BLK, _BLK), 0)
    cio = lax.broadcasted_iota(jnp.int32, (_BLK, _BLK), 1)
    tri = (cio < rio).astype(jnp.float32)
    nblk = s // _BLK
    c_blocks = []
    running = jnp.zeros((1, e), jnp.int32)
    for b in range(nblk):
        rb = oh[b * _BLK:(b + 1) * _BLK, :]
        cb = lax.dot_general(tri, rb, (((1,), (0,)), ((), ())),
                             preferred_element_type=jnp.float32)
        c_blocks.append(cb.astype(jnp.int32) + running)
        running = running + jnp.sum(rb, axis=0, keepdims=True).astype(jnp.int32)
    c = jnp.concatenate(c_blocks, axis=0)

    cnts = [jnp.sum(oh[:, j:j + 1]).astype(jnp.int32) for j in range(e)]
    po = [jnp.int32(0)]
    ct = [jnp.int32(0)]
    for j in range(e):
        ntile_j = (cnts[j] + (_TILE - 1)) // _TILE
        po.append(po[j] + ntile_j * _TILE)
        ct.append(ct[j] + ntile_j)

    po_vec = jnp.concatenate(
        [jnp.zeros((1, 1), jnp.int32) + po[j] for j in range(e)], axis=1)
    p = c + po_vec
    oh0i = oh0.astype(jnp.int32)
    oh1i = oh1.astype(jnp.int32)
    pos0 = jnp.sum(oh0i * p, axis=1, keepdims=True)
    pos1 = jnp.sum(oh1i * p, axis=1, keepdims=True)

    pos_ref[...] = jnp.where(eio == 0, pos0, jnp.where(eio == 1, pos1, 0))
    lio = lax.broadcasted_iota(jnp.int32, (s, 2 * 16), 1)
    gw_ref[...] = jnp.where(lio < 16, w1, 1.0 - w1)

    nt_max = exp_ref.shape[0]
    last = jnp.maximum(ct[e] - 1, 0)
    for i in range(nt_max):
        ii = jnp.minimum(jnp.int32(i), last)
        ex = jnp.int32(0)
        for j in range(1, e):
            ex = ex + (ct[j] <= ii).astype(jnp.int32)
        exp_ref[i] = ex
        xgi_ref[i] = ii
        valid_ref[i] = (jnp.int32(i) < ct[e]).astype(jnp.int32)


def _run_router(x, gate_weight):
    s, _ = x.shape
    e = gate_weight.shape[0]
    nt_max = (s * _K) // _TILE + e - 1
    smem_spec = pl.BlockSpec(memory_space=pltpu.SMEM)
    return pl.pallas_call(
        _router_body,
        out_shape=(
            jax.ShapeDtypeStruct((s, e), jnp.int32),
            jax.ShapeDtypeStruct((s, 32), jnp.float32),
            jax.ShapeDtypeStruct((nt_max,), jnp.int32),
            jax.ShapeDtypeStruct((nt_max,), jnp.int32),
            jax.ShapeDtypeStruct((nt_max,), jnp.int32),
        ),
        out_specs=(pl.BlockSpec(), pl.BlockSpec(),
                   smem_spec, smem_spec, smem_spec),
    )(x, gate_weight)


def _split_pos(pos_v, idx0_v, idx1_v, per, nl, e):
    for ch in range(per // nl):
        flat = (lax.iota(jnp.int32, nl) + ch * nl) * e
        sl = pl.ds(ch * nl, nl)
        idx0_v[sl] = plsc.load_gather(pos_v, [flat])
        idx1_v[sl] = plsc.load_gather(pos_v, [flat + 1])


def _make_sc_dispatch(n_tok, d, e, n_pad):
    info = plsc.get_sparse_core_info()
    nw = info.num_cores * info.num_subcores
    per = n_tok // nw
    mesh = plsc.VectorSubcoreMesh(core_axis_name="c", subcore_axis_name="s")

    @functools.partial(
        pl.kernel,
        out_type=jax.ShapeDtypeStruct((n_pad, d), jnp.float32),
        mesh=mesh,
        compiler_params=pltpu.CompilerParams(needs_layout_passes=False),
        scratch_types=[
            pltpu.VMEM((per * e,), jnp.int32),
            pltpu.VMEM((per,), jnp.int32),
            pltpu.VMEM((per,), jnp.int32),
            pltpu.VMEM((per, d), jnp.float32),
            pltpu.SemaphoreType.DMA,
        ],
    )
    def k(x_hbm, pos_hbm, xg_hbm, pos_v, idx0_v, idx1_v, xloc_v, sem):
        wid = lax.axis_index("s") * info.num_cores + lax.axis_index("c")
        base = wid * per
        pltpu.sync_copy(x_hbm.at[pl.ds(base, per)], xloc_v)
        pltpu.sync_copy(pos_hbm.at[pl.ds(base * e, per * e)], pos_v)
        _split_pos(pos_v, idx0_v, idx1_v, per, info.num_lanes, e)
        cp0 = pltpu.async_copy(xloc_v, xg_hbm.at[idx0_v], sem)
        cp1 = pltpu.async_copy(xloc_v, xg_hbm.at[idx1_v], sem)
        cp0.wait()
        cp1.wait()

    return k


def _make_sc_combine(n_tok, d, e):
    info = plsc.get_sparse_core_info()
    nl = info.num_lanes
    nw = info.num_cores * info.num_subcores
    per = n_tok // nw
    mesh = plsc.VectorSubcoreMesh(core_axis_name="c", subcore_axis_name="s")

    @functools.partial(
        pl.kernel,
        out_type=jax.ShapeDtypeStruct((n_tok, d), jnp.float32),
        mesh=mesh,
        compiler_params=pltpu.CompilerParams(needs_layout_passes=False),
        scratch_types=[
            pltpu.VMEM((per * e,), jnp.int32),
            pltpu.VMEM((per,), jnp.int32),
            pltpu.VMEM((per,), jnp.int32),
            pltpu.VMEM((per, 2 * nl), jnp.float32),
            pltpu.VMEM((per, d), jnp.float32),
            pltpu.VMEM((per, d), jnp.float32),
            pltpu.SemaphoreType.DMA,
            pltpu.SemaphoreType.DMA,
        ],
    )
    def k(h_hbm, pos_hbm, gw_hbm, out_hbm,
          pos_v, idx0_v, idx1_v, gw_v, rows0_v, rows1_v, sem_a, sem_b):
        wid = lax.axis_index("s") * info.num_cores + lax.axis_index("c")
        base = wid * per
        half = per // 2
        pltpu.sync_copy(pos_hbm.at[pl.ds(base * e, per * e)], pos_v)
        _split_pos(pos_v, idx0_v, idx1_v, per, nl, e)
        ha = pl.ds(0, half)
        hb = pl.ds(half, half)
        cpa0 = pltpu.async_copy(h_hbm.at[idx0_v.at[ha]], rows0_v.at[ha], sem_a)
        cpa1 = pltpu.async_copy(h_hbm.at[idx1_v.at[ha]], rows1_v.at[ha], sem_a)
        cpb0 = pltpu.async_copy(h_hbm.at[idx0_v.at[hb]], rows0_v.at[hb], sem_b)
        cpb1 = pltpu.async_copy(h_hbm.at[idx1_v.at[hb]], rows1_v.at[hb], sem_b)
        pltpu.sync_copy(gw_hbm.at[pl.ds(base, per)], gw_v)

        def body(i, carry):
            g0 = gw_v[i, pl.ds(0, nl)]
            g1 = gw_v[i, pl.ds(nl, nl)]
            for cch in range(d // nl):
                sl = pl.ds(cch * nl, nl)
                rows0_v[i, sl] = rows0_v[i, sl] * g0 + rows1_v[i, sl] * g1
            return carry

        cpa0.wait()
        cpa1.wait()
        lax.fori_loop(0, half, body, 0)
        cpb0.wait()
        cpb1.wait()
        lax.fori_loop(half, per, body, 0)
        pltpu.sync_copy(rows0_v, out_hbm.at[pl.ds(base, per)])

    return k


def _gelu_exact(a):
    return 0.5 * a * (1.0 + lax.erf(a * 0.7071067811865476))


def _ffn_body(exp_ref, xgi_ref, valid_ref,
              xg_ref, fc_ref, proj_ref, h_ref):
    p = pl.program_id(0)

    @pl.when(valid_ref[p] == 1)
    def _():
        x = xg_ref[...].astype(jnp.bfloat16)
        a = lax.dot_general(x, fc_ref[0].astype(jnp.bfloat16),
                            (((1,), (1,)), ((), ())),
                            preferred_element_type=jnp.float32)
        g = _gelu_exact(a).astype(jnp.bfloat16)
        h_ref[...] = lax.dot_general(g, proj_ref[0].astype(jnp.bfloat16),
                                     (((1,), (1,)), ((), ())),
                                     preferred_element_type=jnp.float32)


def _run_ffn(tile_exp, tile_xgi, tile_valid, xg, c_fc_weight, c_proj_weight):
    n_rows, d = xg.shape
    e, dff, _ = c_fc_weight.shape
    nt_max = tile_exp.shape[0]
    grid_spec = pltpu.PrefetchScalarGridSpec(
        num_scalar_prefetch=3,
        grid=(nt_max,),
        in_specs=[
            pl.BlockSpec((_TILE, d), lambda p, er, xr, vr: (xr[p], 0)),
            pl.BlockSpec((1, dff, d), lambda p, er, xr, vr: (er[p], 0, 0)),
            pl.BlockSpec((1, d, dff), lambda p, er, xr, vr: (er[p], 0, 0)),
        ],
        out_specs=pl.BlockSpec((_TILE, d), lambda p, er, xr, vr: (xr[p], 0)),
    )
    return pl.pallas_call(
        _ffn_body,
        grid_spec=grid_spec,
        out_shape=jax.ShapeDtypeStruct((n_rows, d), jnp.float32),
        compiler_params=pltpu.CompilerParams(
            dimension_semantics=("arbitrary",)),
    )(tile_exp, tile_xgi, tile_valid, xg, c_fc_weight, c_proj_weight)


def kernel(hidden_states, gate_weight, c_fc_weight, c_proj_weight):
    b, s, d = hidden_states.shape
    e, dff, _ = c_fc_weight.shape
    x = hidden_states.reshape(-1, d)
    n_tok = x.shape[0]
    n_rows = n_tok * _K
    nt = n_rows // _TILE

    pos8, gw, tile_exp, tile_xgi, tile_valid = _run_router(x, gate_weight)

    nt_max = tile_exp.shape[0]
    n_pad = nt_max * _TILE
    pos_flat = pos8.reshape(-1)
    xg = _make_sc_dispatch(n_tok, d, e, n_pad)(x, pos_flat)
    h = _run_ffn(tile_exp, tile_xgi, tile_valid, xg, c_fc_weight, c_proj_weight)
    out = _make_sc_combine(n_tok, d, e)(h, pos_flat, gw)
    return out.reshape(b, s, d)


# speedup vs baseline: 1.0055x; 1.0055x over previous
import functools

import jax
import jax.numpy as jnp
from jax import lax
from jax.experimental import pallas as pl
from jax.experimental.pallas import tpu as pltpu
from jax.experimental.pallas import tpu_sc as plsc

_TILE = 512
_BLK = 128
_K = 2


def _router_body(x_ref, wg_ref, pos_ref, gw_ref,
                 exp_ref, xgi_ref, valid_ref):
    x = x_ref[...]
    wg = wg_ref[...]
    logits = lax.dot_general(x, wg, (((1,), (1,)), ((), ())),
                             preferred_element_type=jnp.float32)
    s, e = logits.shape
    eio = lax.broadcasted_iota(jnp.int32, (s, e), 1)
    m1 = jnp.max(logits, axis=1, keepdims=True)
    i1 = jnp.min(jnp.where(logits == m1, eio, e), axis=1, keepdims=True)
    l2 = jnp.where(eio == i1, -jnp.inf, logits)
    m2 = jnp.max(l2, axis=1, keepdims=True)
    i2 = jnp.min(jnp.where(l2 == m2, eio, e), axis=1, keepdims=True)
    w1 = 1.0 / (1.0 + jnp.exp(m2 - m1))

    oh0 = (eio == i1).astype(jnp.float32)
    oh1 = (eio == i2).astype(jnp.float32)
    oh = oh0 + oh1

    rio = lax.broadcasted_iota(jnp.int32, (_BLK, _BLK), 0)
    cio = lax.broadcasted_iota(jnp.int32, (_BLK, _BLK), 1)
    tri = (cio < rio).astype(jnp.float32)
    nblk = s // _BLK
    c_blocks = []
    running = jnp.zeros((1, e), jnp.int32)
    for b in range(nblk):
        rb = oh[b * _BLK:(b + 1) * _BLK, :]
        cb = lax.dot_general(tri, rb, (((1,), (0,)), ((), ())),
                             preferred_element_type=jnp.float32)
        c_blocks.append(cb.astype(jnp.int32) + running)
        running = running + jnp.sum(rb, axis=0, keepdims=True).astype(jnp.int32)
    c = jnp.concatenate(c_blocks, axis=0)

    cnts = [jnp.sum(oh[:, j:j + 1]).astype(jnp.int32) for j in range(e)]
    po = [jnp.int32(0)]
    ct = [jnp.int32(0)]
    for j in range(e):
        ntile_j = (cnts[j] + (_TILE - 1)) // _TILE
        po.append(po[j] + ntile_j * _TILE)
        ct.append(ct[j] + ntile_j)

    po_vec = jnp.concatenate(
        [jnp.zeros((1, 1), jnp.int32) + po[j] for j in range(e)], axis=1)
    p = c + po_vec
    oh0i = oh0.astype(jnp.int32)
    oh1i = oh1.astype(jnp.int32)
    pos0 = jnp.sum(oh0i * p, axis=1, keepdims=True)
    pos1 = jnp.sum(oh1i * p, axis=1, keepdims=True)

    pos_ref[...] = jnp.where(eio == 0, pos0, jnp.where(eio == 1, pos1, 0))
    lio = lax.broadcasted_iota(jnp.int32, (s, 2 * 16), 1)
    gw_ref[...] = jnp.where(lio < 16, w1, 1.0 - w1)

    nt_max = exp_ref.shape[0]
    last = jnp.maximum(ct[e] - 1, 0)
    for i in range(nt_max):
        ii = jnp.minimum(jnp.int32(i), last)
        ex = jnp.int32(0)
        for j in range(1, e):
            ex = ex + (ct[j] <= ii).astype(jnp.int32)
        exp_ref[i] = ex
        xgi_ref[i] = ii
        valid_ref[i] = (jnp.int32(i) < ct[e]).astype(jnp.int32)


def _run_router(x, gate_weight):
    s, _ = x.shape
    e = gate_weight.shape[0]
    nt_max = (s * _K) // _TILE + e - 1
    smem_spec = pl.BlockSpec(memory_space=pltpu.SMEM)
    return pl.pallas_call(
        _router_body,
        out_shape=(
            jax.ShapeDtypeStruct((s, e), jnp.int32),
            jax.ShapeDtypeStruct((s, 32), jnp.float32),
            jax.ShapeDtypeStruct((nt_max,), jnp.int32),
            jax.ShapeDtypeStruct((nt_max,), jnp.int32),
            jax.ShapeDtypeStruct((nt_max,), jnp.int32),
        ),
        out_specs=(pl.BlockSpec(), pl.BlockSpec(),
                   smem_spec, smem_spec, smem_spec),
    )(x, gate_weight)


def _split_pos(pos_v, idx0_v, idx1_v, per, nl, e):
    for ch in range(per // nl):
        flat = (lax.iota(jnp.int32, nl) + ch * nl) * e
        sl = pl.ds(ch * nl, nl)
        idx0_v[sl] = plsc.load_gather(pos_v, [flat])
        idx1_v[sl] = plsc.load_gather(pos_v, [flat + 1])


def _make_sc_dispatch(n_tok, d, e, n_pad):
    info = plsc.get_sparse_core_info()
    nw = info.num_cores * info.num_subcores
    per = n_tok // nw
    mesh = plsc.VectorSubcoreMesh(core_axis_name="c", subcore_axis_name="s")

    @functools.partial(
        pl.kernel,
        out_type=jax.ShapeDtypeStruct((n_pad, d), jnp.float32),
        mesh=mesh,
        compiler_params=pltpu.CompilerParams(needs_layout_passes=False),
        scratch_types=[
            pltpu.VMEM((per * e,), jnp.int32),
            pltpu.VMEM((per,), jnp.int32),
            pltpu.VMEM((per,), jnp.int32),
            pltpu.VMEM((per, d), jnp.float32),
            pltpu.SemaphoreType.DMA,
        ],
    )
    def k(x_hbm, pos_hbm, xg_hbm, pos_v, idx0_v, idx1_v, xloc_v, sem):
        wid = lax.axis_index("s") * info.num_cores + lax.axis_index("c")
        base = wid * per
        pltpu.sync_copy(x_hbm.at[pl.ds(base, per)], xloc_v)
        pltpu.sync_copy(pos_hbm.at[pl.ds(base * e, per * e)], pos_v)
        _split_pos(pos_v, idx0_v, idx1_v, per, info.num_lanes, e)
        cp0 = pltpu.async_copy(xloc_v, xg_hbm.at[idx0_v], sem)
        cp1 = pltpu.async_copy(xloc_v, xg_hbm.at[idx1_v], sem)
        cp0.wait()
        cp1.wait()

    return k


def _make_sc_combine(n_tok, d, e):
    info = plsc.get_sparse_core_info()
    nl = info.num_lanes
    nw = info.num_cores * info.num_subcores
    per = n_tok // nw
    mesh = plsc.VectorSubcoreMesh(core_axis_name="c", subcore_axis_name="s")

    @functools.partial(
        pl.kernel,
        out_type=jax.ShapeDtypeStruct((n_tok, d), jnp.float32),
        mesh=mesh,
        compiler_params=pltpu.CompilerParams(needs_layout_passes=False),
        scratch_types=[
            pltpu.VMEM((per * e,), jnp.int32),
            pltpu.VMEM((per,), jnp.int32),
            pltpu.VMEM((per,), jnp.int32),
            pltpu.VMEM((per, 2 * nl), jnp.float32),
            pltpu.VMEM((per, d), jnp.float32),
            pltpu.VMEM((per, d), jnp.float32),
            pltpu.SemaphoreType.DMA,
        ],
    )
    def k(h_hbm, pos_hbm, gw_hbm, out_hbm,
          pos_v, idx0_v, idx1_v, gw_v, rows0_v, rows1_v, sem):
        wid = lax.axis_index("s") * info.num_cores + lax.axis_index("c")
        base = wid * per
        pltpu.sync_copy(pos_hbm.at[pl.ds(base * e, per * e)], pos_v)
        _split_pos(pos_v, idx0_v, idx1_v, per, nl, e)
        cp0 = pltpu.async_copy(h_hbm.at[idx0_v], rows0_v, sem)
        cp1 = pltpu.async_copy(h_hbm.at[idx1_v], rows1_v, sem)
        pltpu.sync_copy(gw_hbm.at[pl.ds(base, per)], gw_v)
        cp0.wait()
        cp1.wait()

        def body(i, carry):
            g0 = gw_v[i, pl.ds(0, nl)]
            g1 = gw_v[i, pl.ds(nl, nl)]
            for cch in range(d // nl):
                sl = pl.ds(cch * nl, nl)
                rows0_v[i, sl] = rows0_v[i, sl] * g0 + rows1_v[i, sl] * g1
            return carry

        lax.fori_loop(0, per, body, 0)
        pltpu.sync_copy(rows0_v, out_hbm.at[pl.ds(base, per)])

    return k


def _gelu_exact(a):
    return 0.5 * a * (1.0 + lax.erf(a * 0.7071067811865476))


def _ffn_body(exp_ref, xgi_ref, valid_ref,
              xg_ref, fc_ref, proj_ref, h_ref):
    p = pl.program_id(0)

    @pl.when(valid_ref[p] == 1)
    def _():
        x = xg_ref[...].astype(jnp.bfloat16)
        a = lax.dot_general(x, fc_ref[0].astype(jnp.bfloat16),
                            (((1,), (1,)), ((), ())),
                            preferred_element_type=jnp.float32)
        g = _gelu_exact(a).astype(jnp.bfloat16)
        h_ref[...] = lax.dot_general(g, proj_ref[0].astype(jnp.bfloat16),
                                     (((1,), (1,)), ((), ())),
                                     preferred_element_type=jnp.float32)


def _run_ffn(tile_exp, tile_xgi, tile_valid, xg, c_fc_weight, c_proj_weight):
    n_rows, d = xg.shape
    e, dff, _ = c_fc_weight.shape
    nt_max = tile_exp.shape[0]
    grid_spec = pltpu.PrefetchScalarGridSpec(
        num_scalar_prefetch=3,
        grid=(nt_max,),
        in_specs=[
            pl.BlockSpec((_TILE, d), lambda p, er, xr, vr: (xr[p], 0)),
            pl.BlockSpec((1, dff, d), lambda p, er, xr, vr: (er[p], 0, 0)),
            pl.BlockSpec((1, d, dff), lambda p, er, xr, vr: (er[p], 0, 0)),
        ],
        out_specs=pl.BlockSpec((_TILE, d), lambda p, er, xr, vr: (xr[p], 0)),
    )
    return pl.pallas_call(
        _ffn_body,
        grid_spec=grid_spec,
        out_shape=jax.ShapeDtypeStruct((n_rows, d), jnp.float32),
        compiler_params=pltpu.CompilerParams(
            dimension_semantics=("arbitrary",)),
    )(tile_exp, tile_xgi, tile_valid, xg, c_fc_weight, c_proj_weight)


def kernel(hidden_states, gate_weight, c_fc_weight, c_proj_weight):
    b, s, d = hidden_states.shape
    e, dff, _ = c_fc_weight.shape
    x = hidden_states.reshape(-1, d)
    n_tok = x.shape[0]
    n_rows = n_tok * _K
    nt = n_rows // _TILE

    pos8, gw, tile_exp, tile_xgi, tile_valid = _run_router(x, gate_weight)

    nt_max = tile_exp.shape[0]
    n_pad = nt_max * _TILE
    pos_flat = pos8.reshape(-1)
    xg = _make_sc_dispatch(n_tok, d, e, n_pad)(x, pos_flat)
    h = _run_ffn(tile_exp, tile_xgi, tile_valid, xg, c_fc_weight, c_proj_weight)
    out = _make_sc_combine(n_tok, d, e)(h, pos_flat, gw)
    return out.reshape(b, s, d)
